# baseline (device time: 8451 ns/iter reference)
import jax
import jax.numpy as jnp
from jax import lax
from jax.experimental import pallas as pl
from jax.experimental.pallas import tpu as pltpu

N_DEV = 4


def kernel(x):
    m_per, n = x.shape

    def body(x_hbm, out_hbm, xv_ref, ov_ref, tot_ref, comm_ref,
             send_sems, recv_sems, copy_sems, exit_sem):
        my_pos = lax.axis_index("i")
        left = (my_pos - 1) % N_DEV
        right = (my_pos + 1) % N_DEV

        in_copy = pltpu.make_async_copy(x_hbm, xv_ref, copy_sems.at[0])
        in_copy.start()

        barrier_sem = pltpu.get_barrier_semaphore()
        for nbr in [left, right]:
            pl.semaphore_signal(
                barrier_sem, inc=1,
                device_id=(nbr,), device_id_type=pl.DeviceIdType.MESH,
            )

        in_copy.wait()
        xv = xv_ref[:, :]

        blk = 8
        n_blk = m_per // blk
        z0 = xv.reshape(n_blk, blk, n)

        b = z0[:, 0:4] * z0[:, 4:8]
        b = b[:, 0:2] * b[:, 2:4]
        bt = b[:, 0] * b[:, 1]
        t = bt
        r = n_blk
        while r > 1:
            t = t[: r // 2] * t[r // 2 :]
            r //= 2
        tot_ref[:, :] = t

        def desc(src, dst):
            return pltpu.make_async_remote_copy(
                src_ref=tot_ref,
                dst_ref=comm_ref.at[src],
                send_sem=send_sems.at[dst],
                recv_sem=recv_sems.at[src],
                device_id=(dst,),
                device_id_type=pl.DeviceIdType.MESH,
            )

        pairs = [(s, d) for s in range(N_DEV) for d in range(s + 1, N_DEV)]

        pl.semaphore_wait(barrier_sem, 2)

        for src, dst in pairs:
            @pl.when(my_pos == src)
            def _(src=src, dst=dst):
                desc(src, dst).start()

        z = z0
        d = 1
        while d < blk:
            shifted = jnp.concatenate(
                [jnp.ones((n_blk, d, n), jnp.float32), z[:, : blk - d]],
                axis=1,
            )
            z = z * shifted
            d *= 2

        s = bt
        d = 1
        while d < n_blk:
            shifted = jnp.concatenate(
                [jnp.ones((d, n), jnp.float32), s[: n_blk - d]], axis=0
            )
            s = s * shifted
            d *= 2
        ex = jnp.concatenate(
            [jnp.ones((1, n), jnp.float32), s[: n_blk - 1]], axis=0
        )

        for src, dst in pairs:
            @pl.when(my_pos == dst)
            def _(src=src, dst=dst):
                desc(src, dst).wait_recv()

            @pl.when(my_pos == src)
            def _(src=src, dst=dst):
                desc(src, dst).wait_send()

        for nbr in [left, right]:
            pl.semaphore_signal(
                exit_sem, inc=1,
                device_id=(nbr,), device_id_type=pl.DeviceIdType.MESH,
            )

        prefix = jnp.ones((1, n), jnp.float32)
        for k in range(N_DEV - 1):
            prefix = prefix * jnp.where(k < my_pos, comm_ref[k, :, :], 1.0)

        full_ex = ex * prefix
        ov_ref[:, :] = (z * full_ex[:, None, :]).reshape(m_per, n)

        out_copy = pltpu.make_async_copy(ov_ref, out_hbm, copy_sems.at[1])
        out_copy.start()
        out_copy.wait()
        pl.semaphore_wait(exit_sem, 2)

    return pl.pallas_call(
        body,
        out_shape=jax.ShapeDtypeStruct((m_per, n), x.dtype),
        in_specs=[pl.BlockSpec(memory_space=pl.ANY)],
        out_specs=pl.BlockSpec(memory_space=pl.ANY),
        scratch_shapes=[
            pltpu.VMEM((m_per, n), jnp.float32),
            pltpu.VMEM((m_per, n), jnp.float32),
            pltpu.VMEM((1, n), jnp.float32),
            pltpu.VMEM((N_DEV, 1, n), jnp.float32),
            pltpu.SemaphoreType.DMA((N_DEV,)),
            pltpu.SemaphoreType.DMA((N_DEV,)),
            pltpu.SemaphoreType.DMA((2,)),
            pltpu.SemaphoreType.REGULAR,
        ],
        compiler_params=pltpu.CompilerParams(collective_id=0),
    )(x)


# device time: 8426 ns/iter; 1.0030x vs baseline; 1.0030x over previous
import jax
import jax.numpy as jnp
from jax import lax
from jax.experimental import pallas as pl
from jax.experimental.pallas import tpu as pltpu

N_DEV = 4


def kernel(x):
    m_per, n = x.shape

    def body(x_hbm, out_hbm, xv_ref, ov_ref, tot_ref, comm_ref,
             send_sems, recv_sems, copy_sems, exit_sem):
        my_pos = lax.axis_index("i")
        left = (my_pos - 1) % N_DEV
        right = (my_pos + 1) % N_DEV

        in_copy = pltpu.make_async_copy(x_hbm, xv_ref, copy_sems.at[0])
        in_copy.start()

        barrier_sem = pltpu.get_barrier_semaphore()
        for nbr in [left, right]:
            pl.semaphore_signal(
                barrier_sem, inc=1,
                device_id=(nbr,), device_id_type=pl.DeviceIdType.MESH,
            )

        in_copy.wait()
        xv = xv_ref[:, :]

        blk = 8
        n_blk = m_per // blk
        z0 = xv.reshape(n_blk, blk, n)

        b = z0[:, 0:4] * z0[:, 4:8]
        b = b[:, 0:2] * b[:, 2:4]
        bt = b[:, 0] * b[:, 1]
        t = bt
        r = n_blk
        while r > 1:
            t = t[: r // 2] * t[r // 2 :]
            r //= 2
        tot_ref[:, :] = t

        def desc(src, dst):
            return pltpu.make_async_remote_copy(
                src_ref=tot_ref,
                dst_ref=comm_ref.at[src],
                send_sem=send_sems.at[dst],
                recv_sem=recv_sems.at[src],
                device_id=(dst,),
                device_id_type=pl.DeviceIdType.MESH,
            )

        pairs = [(s, d) for s in range(N_DEV) for d in range(s + 1, N_DEV)]

        pl.semaphore_wait(barrier_sem, 2)

        for src, dst in pairs:
            @pl.when(my_pos == src)
            def _(src=src, dst=dst):
                desc(src, dst).start()

        z = z0
        d = 1
        while d < blk:
            shifted = jnp.concatenate(
                [jnp.ones((n_blk, d, n), jnp.float32), z[:, : blk - d]],
                axis=1,
            )
            z = z * shifted
            d *= 2

        s = bt
        d = 1
        while d < n_blk:
            shifted = jnp.concatenate(
                [jnp.ones((d, n), jnp.float32), s[: n_blk - d]], axis=0
            )
            s = s * shifted
            d *= 2
        ex = jnp.concatenate(
            [jnp.ones((1, n), jnp.float32), s[: n_blk - 1]], axis=0
        )

        for src, dst in pairs:
            @pl.when(my_pos == dst)
            def _(src=src, dst=dst):
                desc(src, dst).wait_recv()

            @pl.when(my_pos == src)
            def _(src=src, dst=dst):
                desc(src, dst).wait_send()

        for nbr in [left, right]:
            pl.semaphore_signal(
                exit_sem, inc=1,
                device_id=(nbr,), device_id_type=pl.DeviceIdType.MESH,
            )

        prefix = jnp.ones((1, n), jnp.float32)
        for k in range(N_DEV - 1):
            prefix = prefix * jnp.where(k < my_pos, comm_ref[k, :, :], 1.0)

        full_ex = ex * prefix
        ov_ref[:, :] = (z * full_ex[:, None, :]).reshape(m_per, n)

        out_copy = pltpu.make_async_copy(ov_ref, out_hbm, copy_sems.at[1])
        out_copy.start()
        out_copy.wait()
        pl.semaphore_wait(exit_sem, 2)

    return pl.pallas_call(
        body,
        out_shape=jax.ShapeDtypeStruct((m_per, n), x.dtype),
        in_specs=[pl.BlockSpec(memory_space=pltpu.MemorySpace.HBM)],
        out_specs=pl.BlockSpec(memory_space=pltpu.MemorySpace.HBM),
        scratch_shapes=[
            pltpu.VMEM((m_per, n), jnp.float32),
            pltpu.VMEM((m_per, n), jnp.float32),
            pltpu.VMEM((1, n), jnp.float32),
            pltpu.VMEM((N_DEV, 1, n), jnp.float32),
            pltpu.SemaphoreType.DMA((N_DEV,)),
            pltpu.SemaphoreType.DMA((N_DEV,)),
            pltpu.SemaphoreType.DMA((2,)),
            pltpu.SemaphoreType.REGULAR,
        ],
        compiler_params=pltpu.CompilerParams(collective_id=0),
    )(x)


# device time: 8358 ns/iter; 1.0111x vs baseline; 1.0081x over previous
import functools

import jax
import jax.numpy as jnp
from jax import lax
from jax.experimental import pallas as pl
from jax.experimental.pallas import tpu as pltpu

N_DEV = 4


def kernel(x):
    m_per, n = x.shape

    def body(x_ref, out_ref, tot_ref, comm_ref, send_sems, recv_sems,
             exit_sem):
        my_pos = lax.axis_index("i")
        left = (my_pos - 1) % N_DEV
        right = (my_pos + 1) % N_DEV

        barrier_sem = pltpu.get_barrier_semaphore()
        for nbr in [left, right]:
            pl.semaphore_signal(
                barrier_sem, inc=1,
                device_id=(nbr,), device_id_type=pl.DeviceIdType.MESH,
            )

        xv = x_ref[:, :]

        blk = 8
        n_blk = m_per // blk
        z0 = xv.reshape(n_blk, blk, n)

        b = z0[:, 0:4] * z0[:, 4:8]
        b = b[:, 0:2] * b[:, 2:4]
        bt = b[:, 0] * b[:, 1]
        t = bt
        r = n_blk
        while r > 1:
            t = t[: r // 2] * t[r // 2 :]
            r //= 2
        tot_ref[:, :] = t

        def desc(src, dst):
            return pltpu.make_async_remote_copy(
                src_ref=tot_ref,
                dst_ref=comm_ref.at[src],
                send_sem=send_sems.at[dst],
                recv_sem=recv_sems.at[src],
                device_id=(dst,),
                device_id_type=pl.DeviceIdType.MESH,
            )

        pairs = [(s, d) for s in range(N_DEV) for d in range(s + 1, N_DEV)]

        pl.semaphore_wait(barrier_sem, 2)

        for src, dst in pairs:
            @pl.when(my_pos == src)
            def _(src=src, dst=dst):
                desc(src, dst).start()

        z = z0
        d = 1
        while d < blk:
            shifted = jnp.concatenate(
                [jnp.ones((n_blk, d, n), jnp.float32), z[:, : blk - d]],
                axis=1,
            )
            z = z * shifted
            d *= 2

        s = bt
        d = 1
        while d < n_blk:
            shifted = jnp.concatenate(
                [jnp.ones((d, n), jnp.float32), s[: n_blk - d]], axis=0
            )
            s = s * shifted
            d *= 2
        ex = jnp.concatenate(
            [jnp.ones((1, n), jnp.float32), s[: n_blk - 1]], axis=0
        )

        for src, dst in pairs:
            @pl.when(my_pos == dst)
            def _(src=src, dst=dst):
                desc(src, dst).wait_recv()

            @pl.when(my_pos == src)
            def _(src=src, dst=dst):
                desc(src, dst).wait_send()

        for nbr in [left, right]:
            pl.semaphore_signal(
                exit_sem, inc=1,
                device_id=(nbr,), device_id_type=pl.DeviceIdType.MESH,
            )

        prefix = jnp.ones((1, n), jnp.float32)
        for k in range(N_DEV - 1):
            prefix = prefix * jnp.where(k < my_pos, comm_ref[k, :, :], 1.0)

        full_ex = ex * prefix
        out_ref[:, :] = (z * full_ex[:, None, :]).reshape(m_per, n)

        pl.semaphore_wait(exit_sem, 2)

    return pl.pallas_call(
        body,
        out_shape=jax.ShapeDtypeStruct((m_per, n), x.dtype),
        in_specs=[pl.BlockSpec(memory_space=pltpu.VMEM)],
        out_specs=pl.BlockSpec(memory_space=pltpu.VMEM),
        scratch_shapes=[
            pltpu.VMEM((1, n), jnp.float32),
            pltpu.VMEM((N_DEV, 1, n), jnp.float32),
            pltpu.SemaphoreType.DMA((N_DEV,)),
            pltpu.SemaphoreType.DMA((N_DEV,)),
            pltpu.SemaphoreType.REGULAR,
        ],
        compiler_params=pltpu.CompilerParams(collective_id=0),
    )(x)
